# baseline (speedup 1.0000x reference)
import functools

import jax
import jax.numpy as jnp
from jax.experimental import pallas as pl
from jax.experimental.pallas import tpu as pltpu

BM = 200
NB = 50
SB = 0
NS = NB - SB


def _u_body(x_ref, w1_ref, b1_ref, u_ref):
    u_ref[:] = (
        jnp.dot(x_ref[:], w1_ref[:], preferred_element_type=jnp.float32)
        + b1_ref[:]
    )


def _gcn_body(u_ref, w2_ref, b2_ref, adj_ref, out_ref, zf_scr, zb_scr, stash_scr):
    p = pl.program_id(0)
    i = pl.program_id(1)

    @pl.when(p == 0)
    def _phase0():
        pp = jnp.dot(adj_ref[:], u_ref[:], preferred_element_type=jnp.float32)
        zblk = (
            jnp.dot(
                jnp.maximum(pp, 0.0), w2_ref[:], preferred_element_type=jnp.float32
            )
            + b2_ref[:]
        )
        zf_scr[pl.ds(i * BM, BM), :] = zblk
        zb_scr[pl.ds(i * BM, BM), :] = zblk.astype(jnp.bfloat16)

        @pl.when(i < SB)
        def _stash():
            stash_scr[pl.ds(i * BM, BM), :] = adj_ref[:].astype(jnp.bfloat16)

    @pl.when(p == 1)
    def _phase1():
        @pl.when(i < NS)
        def _streamed():
            out_ref[:] = jnp.dot(
                adj_ref[:], zf_scr[:], preferred_element_type=jnp.float32
            )

        @pl.when(i >= NS)
        def _stashed():
            k = i - NS
            a = stash_scr[pl.ds(k * BM, BM), :]
            out_ref[:] = jnp.dot(a, zb_scr[:], preferred_element_type=jnp.float32)


@jax.jit
def kernel(x, adj, W1, b1, W2, b2):
    n, din = x.shape
    dh = W1.shape[1]
    dout = W2.shape[1]

    u = pl.pallas_call(
        _u_body,
        out_shape=jax.ShapeDtypeStruct((n, dh), jnp.float32),
    )(x, W1, b1.reshape(1, dh))

    def adj_map(p, i):
        return (jnp.where(p == 0, i, jnp.minimum(SB + i, NB - 1)), 0)

    def out_map(p, i):
        return (
            jnp.where(p == 0, 0, jnp.where(i < NS, SB + i, i - NS)),
            0,
        )

    out = pl.pallas_call(
        _gcn_body,
        grid=(2, NB),
        in_specs=[
            pl.BlockSpec((n, dh), lambda p, i: (0, 0)),
            pl.BlockSpec((dh, dout), lambda p, i: (0, 0)),
            pl.BlockSpec((1, dout), lambda p, i: (0, 0)),
            pl.BlockSpec((BM, n), adj_map),
        ],
        out_specs=pl.BlockSpec((BM, dout), out_map),
        out_shape=jax.ShapeDtypeStruct((n, dout), jnp.float32),
        scratch_shapes=[
            pltpu.VMEM((n, dout), jnp.float32),
            pltpu.VMEM((n, dout), jnp.bfloat16),
            pltpu.VMEM((SB * BM, n), jnp.bfloat16),
        ],
    )(u, W2, b2.reshape(1, dout), adj)

    return out


# speedup vs baseline: 1.0226x; 1.0226x over previous
import functools

import jax
import jax.numpy as jnp
from jax.experimental import pallas as pl
from jax.experimental.pallas import tpu as pltpu

BM = 200
NB = 50
SB = 7
NS = NB - SB


def _u_body(x_ref, w1_ref, b1_ref, u_ref):
    u_ref[:] = (
        jnp.dot(x_ref[:], w1_ref[:], preferred_element_type=jnp.float32)
        + b1_ref[:]
    )


def _gcn_body(u_ref, w2_ref, b2_ref, adj_ref, out_ref, zf_scr, stash_scr):
    p = pl.program_id(0)
    i = pl.program_id(1)

    @pl.when(p == 0)
    def _phase0():
        pp = jnp.dot(adj_ref[:], u_ref[:], preferred_element_type=jnp.float32)
        zf_scr[pl.ds(i * BM, BM), :] = (
            jnp.dot(
                jnp.maximum(pp, 0.0), w2_ref[:], preferred_element_type=jnp.float32
            )
            + b2_ref[:]
        )

        @pl.when(i < SB)
        def _stash():
            stash_scr[i] = adj_ref[:].astype(jnp.bfloat16)

    @pl.when(p == 1)
    def _phase1():
        @pl.when(i < NS)
        def _streamed():
            out_ref[:] = jnp.dot(
                adj_ref[:], zf_scr[:], preferred_element_type=jnp.float32
            )

        @pl.when(i >= NS)
        def _stashed():
            k = i - NS
            out_ref[:] = jnp.dot(
                stash_scr[k],
                zf_scr[:].astype(jnp.bfloat16),
                preferred_element_type=jnp.float32,
            )


@jax.jit
def kernel(x, adj, W1, b1, W2, b2):
    n, din = x.shape
    dh = W1.shape[1]
    dout = W2.shape[1]

    u = pl.pallas_call(
        _u_body,
        out_shape=jax.ShapeDtypeStruct((n, dh), jnp.float32),
    )(x, W1, b1.reshape(1, dh))

    def adj_map(p, i):
        return (jnp.where(p == 0, i, jnp.minimum(SB + i, NB - 1)), 0)

    def out_map(p, i):
        return (
            jnp.where(p == 0, 0, jnp.where(i < NS, SB + i, i - NS)),
            0,
        )

    out = pl.pallas_call(
        _gcn_body,
        grid=(2, NB),
        in_specs=[
            pl.BlockSpec((n, dh), lambda p, i: (0, 0)),
            pl.BlockSpec((dh, dout), lambda p, i: (0, 0)),
            pl.BlockSpec((1, dout), lambda p, i: (0, 0)),
            pl.BlockSpec((BM, n), adj_map),
        ],
        out_specs=pl.BlockSpec((BM, dout), out_map),
        out_shape=jax.ShapeDtypeStruct((n, dout), jnp.float32),
        scratch_shapes=[
            pltpu.VMEM((n, dout), jnp.float32),
            pltpu.VMEM((SB, BM, n), jnp.bfloat16),
        ],
    )(u, W2, b2.reshape(1, dout), adj)

    return out


# speedup vs baseline: 1.0378x; 1.0149x over previous
import functools

import jax
import jax.numpy as jnp
from jax.experimental import pallas as pl
from jax.experimental.pallas import tpu as pltpu

BM = 200
NB = 50
SB = 6
NS = NB - SB


def _gcn_body(
    x_ref,
    w1_ref,
    b1_ref,
    w2_ref,
    b2_ref,
    adj_ref,
    out_ref,
    u_scr,
    ub_scr,
    zf_scr,
    stash_scr,
):
    p = pl.program_id(0)
    i = pl.program_id(1)

    @pl.when((p == 0) & (i == 0))
    def _compute_u():
        u = (
            jnp.dot(x_ref[:], w1_ref[:], preferred_element_type=jnp.float32)
            + b1_ref[:]
        )
        u_scr[:] = u
        ub_scr[:] = u.astype(jnp.bfloat16)

    @pl.when(p == 0)
    def _phase0():
        @pl.when(i < SB)
        def _stash():
            a_bf = adj_ref[:].astype(jnp.bfloat16)
            stash_scr[i] = a_bf
            pp = jnp.dot(a_bf, ub_scr[:], preferred_element_type=jnp.float32)
            zf_scr[pl.ds(i * BM, BM), :] = (
                jnp.dot(
                    jnp.maximum(pp, 0.0),
                    w2_ref[:],
                    preferred_element_type=jnp.float32,
                )
                + b2_ref[:]
            )

        @pl.when(i >= SB)
        def _nostash():
            pp = jnp.dot(adj_ref[:], u_scr[:], preferred_element_type=jnp.float32)
            zf_scr[pl.ds(i * BM, BM), :] = (
                jnp.dot(
                    jnp.maximum(pp, 0.0),
                    w2_ref[:],
                    preferred_element_type=jnp.float32,
                )
                + b2_ref[:]
            )

    @pl.when(p == 1)
    def _phase1():
        @pl.when(i < NS)
        def _streamed():
            out_ref[:] = jnp.dot(
                adj_ref[:], zf_scr[:], preferred_element_type=jnp.float32
            )

        @pl.when(i >= NS)
        def _stashed():
            k = i - NS
            out_ref[:] = jnp.dot(
                stash_scr[k],
                zf_scr[:].astype(jnp.bfloat16),
                preferred_element_type=jnp.float32,
            )


@jax.jit
def kernel(x, adj, W1, b1, W2, b2):
    n, din = x.shape
    dh = W1.shape[1]
    dout = W2.shape[1]

    def adj_map(p, i):
        return (jnp.where(p == 0, i, jnp.minimum(SB + i, NB - 1)), 0)

    def out_map(p, i):
        return (
            jnp.where(p == 0, SB, jnp.where(i < NS, SB + i, i - NS)),
            0,
        )

    out = pl.pallas_call(
        _gcn_body,
        grid=(2, NB),
        in_specs=[
            pl.BlockSpec((n, din), lambda p, i: (0, 0)),
            pl.BlockSpec((din, dh), lambda p, i: (0, 0)),
            pl.BlockSpec((1, dh), lambda p, i: (0, 0)),
            pl.BlockSpec((dh, dout), lambda p, i: (0, 0)),
            pl.BlockSpec((1, dout), lambda p, i: (0, 0)),
            pl.BlockSpec((BM, n), adj_map),
        ],
        out_specs=pl.BlockSpec((BM, dout), out_map),
        out_shape=jax.ShapeDtypeStruct((n, dout), jnp.float32),
        scratch_shapes=[
            pltpu.VMEM((n, dh), jnp.float32),
            pltpu.VMEM((n, dh), jnp.bfloat16),
            pltpu.VMEM((n, dout), jnp.float32),
            pltpu.VMEM((SB, BM, n), jnp.bfloat16),
        ],
    )(x, W1, b1.reshape(1, dh), W2, b2.reshape(1, dout), adj)

    return out
